# Initial kernel scaffold; baseline (speedup 1.0000x reference)
import jax
import jax.numpy as jnp
from jax.experimental import pallas as pl


def kernel(movie_ids, genres, movie_emb, genre_emb, W1, b1, W2, b2, W3, b3):
    raise NotImplementedError("write your pallas kernel here")


# speedup vs baseline: 4.4010x; 4.4010x over previous
import functools

import jax
import jax.numpy as jnp
from jax import lax
from jax.experimental import pallas as pl
from jax.experimental.pallas import tpu as pltpu
from jax.experimental.pallas import tpu_sc as plsc


def _make_sc_gather(V, D, B):
    info = plsc.get_sparse_core_info()
    NC, NS = info.num_cores, info.num_subcores
    NW = NC * NS
    assert B % (8 * NW) == 0
    b_per_w = B // NW
    mesh = plsc.VectorSubcoreMesh(core_axis_name="c", subcore_axis_name="s")

    @functools.partial(
        pl.kernel,
        mesh=mesh,
        out_type=jax.ShapeDtypeStruct((B, D), jnp.float32),
        scratch_types=[
            pltpu.VMEM((b_per_w,), jnp.int32),
            pltpu.VMEM((b_per_w, D), jnp.float32),
            pltpu.SemaphoreType.DMA,
        ],
    )
    def gather_kernel(table_hbm, idx_hbm, out_hbm, idx_v, rows_v, sem):
        wid = lax.axis_index("s") * NC + lax.axis_index("c")
        base = wid * b_per_w
        pltpu.sync_copy(idx_hbm.at[pl.ds(base, b_per_w)], idx_v)
        pltpu.async_copy(table_hbm.at[idx_v], rows_v, sem).wait()
        pltpu.sync_copy(rows_v, out_hbm.at[pl.ds(base, b_per_w)])

    return gather_kernel


def _mlp_body(mv_ref, gen_ref, ge_ref, w1m_ref, w1g_ref, b1_ref,
              w2_ref, b2_ref, w3_ref, b3_ref, out_ref):
    mv = mv_ref[...]
    gen = gen_ref[...]
    blk = mv.shape[0]
    g = ge_ref.shape[0]

    giota = lax.broadcasted_iota(jnp.int32, (blk, g), 1)
    oh = jnp.zeros((blk, g), jnp.float32)
    for j in range(gen.shape[1]):
        oh = oh + (gen[:, j:j + 1] == giota).astype(jnp.float32)
    ge = ge_ref[...]
    row0 = lax.broadcasted_iota(jnp.int32, ge.shape, 0)
    ge = jnp.where(row0 == 0, 0.0, ge)
    gv = lax.dot_general(oh, ge, (((1,), (0,)), ((), ())),
                         preferred_element_type=jnp.float32)
    gv = gv * (1.0 / gen.shape[1])

    h = lax.dot_general(mv, w1m_ref[...], (((1,), (1,)), ((), ())),
                        preferred_element_type=jnp.float32)
    h = h + lax.dot_general(gv, w1g_ref[...], (((1,), (1,)), ((), ())),
                            preferred_element_type=jnp.float32)
    h = jnp.maximum(h + b1_ref[...], 0.0)
    h = lax.dot_general(h, w2_ref[...], (((1,), (1,)), ((), ())),
                        preferred_element_type=jnp.float32)
    h = jnp.maximum(h + b2_ref[...], 0.0)
    h = lax.dot_general(h, w3_ref[...], (((1,), (1,)), ((), ())),
                        preferred_element_type=jnp.float32)
    h = jnp.maximum(h + b3_ref[...], 0.0)
    ssum = jnp.sum(h * h, axis=1, keepdims=True)
    out_ref[...] = h * (1.0 / jnp.maximum(jnp.sqrt(ssum), 1e-12))


def _mlp_call(movie_vec, genres, genre_emb, w1m, w1g, b1, W2, b2, W3, b3,
              blk=512, interpret=False):
    B, D = movie_vec.shape
    NG = genres.shape[1]
    G = genre_emb.shape[0]
    H1, H2, H3 = W2.shape[1], W3.shape[1], W3.shape[0]
    fixed = lambda i: (0, 0)
    return pl.pallas_call(
        _mlp_body,
        grid=(B // blk,),
        in_specs=[
            pl.BlockSpec((blk, D), lambda i: (i, 0)),
            pl.BlockSpec((blk, NG), lambda i: (i, 0)),
            pl.BlockSpec((G, D), fixed),
            pl.BlockSpec((H1, D), fixed),
            pl.BlockSpec((H1, D), fixed),
            pl.BlockSpec((1, H1), fixed),
            pl.BlockSpec((H2, H1), fixed),
            pl.BlockSpec((1, H2), fixed),
            pl.BlockSpec((H3, H2), fixed),
            pl.BlockSpec((1, H3), fixed),
        ],
        out_specs=pl.BlockSpec((blk, H3), lambda i: (i, 0)),
        out_shape=jax.ShapeDtypeStruct((B, H3), jnp.float32),
        interpret=interpret,
    )(movie_vec, genres, genre_emb, w1m, w1g, b1, W2, b2, W3, b3)


def kernel(movie_ids, genres, movie_emb, genre_emb, W1, b1, W2, b2, W3, b3):
    B = movie_ids.shape[0]
    V, D = movie_emb.shape
    ids = movie_ids.astype(jnp.int32)
    movie_vec = _make_sc_gather(V, D, B)(movie_emb, ids)
    w1m = W1[:, :D]
    w1g = W1[:, D:]
    return _mlp_call(movie_vec, genres.astype(jnp.int32), genre_emb,
                     w1m, w1g, b1.reshape(1, -1), W2, b2.reshape(1, -1),
                     W3, b3.reshape(1, -1))
